# Initial kernel scaffold; baseline (speedup 1.0000x reference)
import jax
import jax.numpy as jnp
from jax.experimental import pallas as pl


def kernel(inputs, token_table, pos_table):
    raise NotImplementedError("write your pallas kernel here")


# speedup vs baseline: 4.8120x; 4.8120x over previous
import functools

import jax
import jax.numpy as jnp
from jax import lax
from jax.experimental import pallas as pl
from jax.experimental.pallas import tpu as pltpu
from jax.experimental.pallas import tpu_sc as plsc

EMBED = 128
VOCAB = 14
SEQ = 64
FUSED_ROWS = SEQ * VOCAB
NC = 2
NS = 16
NW = NC * NS
CHUNK = 128


def _prep_tc(flat, token_table, pos_table):
    b = flat.shape[0]

    def body(flat_ref, tok_ref, pos_ref, f_ref, idx_ref):
        f_ref[...] = pos_ref[...][:, None, :] + tok_ref[...][None, :, :]
        pos_ids = lax.broadcasted_iota(jnp.int32, (b, SEQ), 1)
        idx_ref[...] = flat_ref[...] + pos_ids * VOCAB

    return pl.pallas_call(
        body,
        out_shape=(
            jax.ShapeDtypeStruct((SEQ, VOCAB, EMBED), jnp.float32),
            jax.ShapeDtypeStruct((b, SEQ), jnp.int32),
        ),
    )(flat, token_table, pos_table)


def _gather_sc(fused, idx):
    n_rows = idx.shape[0]
    rows_per_w = n_rows // NW
    n_chunks = rows_per_w // CHUNK
    mesh = plsc.VectorSubcoreMesh(core_axis_name="c", subcore_axis_name="s")

    @functools.partial(
        pl.kernel,
        out_type=jax.ShapeDtypeStruct((n_rows, EMBED), jnp.float32),
        mesh=mesh,
        scratch_types=[
            pltpu.VMEM((CHUNK,), jnp.int32),
            pltpu.VMEM((CHUNK, EMBED), jnp.float32),
            pltpu.SemaphoreType.DMA,
        ],
    )
    def k(fused_hbm, idx_hbm, out_hbm, idx_v, rows_v, sem):
        wid = lax.axis_index("s") * NC + lax.axis_index("c")
        base0 = wid * rows_per_w

        def body(i, carry):
            base = base0 + i * CHUNK
            pltpu.sync_copy(idx_hbm.at[pl.ds(base, CHUNK)], idx_v)
            pltpu.async_copy(fused_hbm.at[idx_v], rows_v, sem).wait()
            pltpu.sync_copy(rows_v, out_hbm.at[pl.ds(base, CHUNK)])
            return carry

        lax.fori_loop(0, n_chunks, body, 0)

    return k(fused, idx)


def kernel(inputs, token_table, pos_table):
    batch = inputs.shape[0]
    flat = inputs.reshape(batch, SEQ)
    fused3, idx = _prep_tc(flat, token_table, pos_table)
    out2 = _gather_sc(fused3.reshape(FUSED_ROWS, EMBED), idx.reshape(-1))
    return out2.reshape(batch, SEQ, EMBED)


# speedup vs baseline: 5.6915x; 1.1828x over previous
import functools

import jax
import jax.numpy as jnp
from jax import lax
from jax.experimental import pallas as pl
from jax.experimental.pallas import tpu as pltpu
from jax.experimental.pallas import tpu_sc as plsc

EMBED = 128
VOCAB = 14
SEQ = 64
FUSED_ROWS = SEQ * VOCAB
NC = 2
NS = 16
NW = NC * NS
CHUNK = 128


def _prep_tc(flat, token_table, pos_table):
    b = flat.shape[0]

    def body(flat_ref, tok_ref, pos_ref, f_ref, idx_ref):
        f_ref[...] = pos_ref[...][:, None, :] + tok_ref[...][None, :, :]
        pos_ids = lax.broadcasted_iota(jnp.int32, (b, SEQ), 1)
        idx_ref[...] = flat_ref[...] + pos_ids * VOCAB

    return pl.pallas_call(
        body,
        out_shape=(
            jax.ShapeDtypeStruct((SEQ, VOCAB, EMBED), jnp.float32),
            jax.ShapeDtypeStruct((b, SEQ), jnp.int32),
        ),
    )(flat, token_table, pos_table)


def _gather_sc(fused, idx):
    n_rows = idx.shape[0]
    rows_per_w = n_rows // NW
    n_chunks = rows_per_w // CHUNK
    mesh = plsc.VectorSubcoreMesh(core_axis_name="c", subcore_axis_name="s")

    @functools.partial(
        pl.kernel,
        out_type=jax.ShapeDtypeStruct((n_rows, EMBED), jnp.float32),
        mesh=mesh,
        scratch_types=[
            pltpu.VMEM((rows_per_w,), jnp.int32),
            pltpu.VMEM((CHUNK, EMBED), jnp.float32),
            pltpu.VMEM((CHUNK, EMBED), jnp.float32),
            pltpu.SemaphoreType.DMA,
            pltpu.SemaphoreType.DMA,
            pltpu.SemaphoreType.DMA,
            pltpu.SemaphoreType.DMA,
        ],
    )
    def k(fused_hbm, idx_hbm, out_hbm, idx_all, rows_v0, rows_v1,
          sem_g0, sem_g1, sem_s0, sem_s1):
        wid = lax.axis_index("s") * NC + lax.axis_index("c")
        base0 = wid * rows_per_w
        pltpu.sync_copy(idx_hbm.at[pl.ds(base0, rows_per_w)], idx_all)

        bufs = ((rows_v0, sem_g0, sem_s0), (rows_v1, sem_g1, sem_s1))

        def body(j, carry):
            for b, (rows_v, sem_g, sem_s) in enumerate(bufs):
                i = j * 2 + b
                base = base0 + i * CHUNK

                @pl.when(j > 0)
                def _():
                    pltpu.make_async_copy(
                        rows_v, out_hbm.at[pl.ds(base0, CHUNK)], sem_s).wait()

                pltpu.async_copy(
                    fused_hbm.at[idx_all.at[pl.ds(i * CHUNK, CHUNK)]],
                    rows_v, sem_g).wait()
                pltpu.async_copy(rows_v, out_hbm.at[pl.ds(base, CHUNK)], sem_s)
            return carry

        lax.fori_loop(0, n_chunks // 2, body, 0)
        for rows_v, _, sem_s in bufs:
            pltpu.make_async_copy(
                rows_v, out_hbm.at[pl.ds(base0, CHUNK)], sem_s).wait()

    return k(fused, idx)


def kernel(inputs, token_table, pos_table):
    batch = inputs.shape[0]
    flat = inputs.reshape(batch, SEQ)
    fused3, idx = _prep_tc(flat, token_table, pos_table)
    out2 = _gather_sc(fused3.reshape(FUSED_ROWS, EMBED), idx.reshape(-1))
    return out2.reshape(batch, SEQ, EMBED)


# speedup vs baseline: 14.6984x; 2.5825x over previous
import functools

import jax
import jax.numpy as jnp
from jax import lax
from jax.experimental import pallas as pl
from jax.experimental.pallas import tpu as pltpu
from jax.experimental.pallas import tpu_sc as plsc

EMBED = 128
VOCAB = 14
SEQ = 64
FUSED_ROWS = SEQ * VOCAB
NC = 2
NS = 16
NW = NC * NS
CHUNK = 128


def _prep_tc(flat, token_table, pos_table):
    b = flat.shape[0]

    def body(flat_ref, tok_ref, pos_ref, f_ref, idx_ref):
        f_ref[...] = pos_ref[...][:, None, :] + tok_ref[...][None, :, :]
        pos_ids = lax.broadcasted_iota(jnp.int32, (b, SEQ), 1)
        idx_ref[...] = flat_ref[...] + pos_ids * VOCAB

    return pl.pallas_call(
        body,
        out_shape=(
            jax.ShapeDtypeStruct((SEQ, VOCAB, EMBED), jnp.float32),
            jax.ShapeDtypeStruct((b, SEQ), jnp.int32),
        ),
    )(flat, token_table, pos_table)


def _gather_sc(fused, idx):
    n_rows = idx.shape[0]
    rows_per_w = n_rows // NW
    n_chunks = rows_per_w // CHUNK
    mesh = plsc.VectorSubcoreMesh(core_axis_name="c", subcore_axis_name="s")

    @functools.partial(
        pl.kernel,
        out_type=jax.ShapeDtypeStruct((n_rows, EMBED), jnp.float32),
        mesh=mesh,
        scratch_types=[
            pltpu.VMEM((rows_per_w,), jnp.int32),
            pltpu.VMEM((CHUNK, EMBED), jnp.float32),
            pltpu.VMEM((CHUNK, EMBED), jnp.float32),
            pltpu.VMEM_SHARED((FUSED_ROWS, EMBED), jnp.float32),
            pltpu.SemaphoreType.DMA,
            pltpu.SemaphoreType.DMA,
            pltpu.SemaphoreType.DMA,
            pltpu.SemaphoreType.DMA,
        ],
    )
    def k(fused_hbm, idx_hbm, out_hbm, idx_all, rows_v0, rows_v1, fused_sp,
          sem_g0, sem_g1, sem_s0, sem_s1):
        sid = lax.axis_index("s")
        wid = sid * NC + lax.axis_index("c")
        base0 = wid * rows_per_w

        @pl.when(sid == 0)
        def _():
            pltpu.sync_copy(fused_hbm, fused_sp)

        pltpu.sync_copy(idx_hbm.at[pl.ds(base0, rows_per_w)], idx_all)
        plsc.subcore_barrier()

        bufs = ((rows_v0, sem_g0, sem_s0), (rows_v1, sem_g1, sem_s1))

        def body(j, carry):
            for b, (rows_v, sem_g, sem_s) in enumerate(bufs):
                i = j * 2 + b
                base = base0 + i * CHUNK

                @pl.when(j > 0)
                def _():
                    pltpu.make_async_copy(
                        rows_v, out_hbm.at[pl.ds(base0, CHUNK)], sem_s).wait()

                pltpu.async_copy(
                    fused_sp.at[idx_all.at[pl.ds(i * CHUNK, CHUNK)]],
                    rows_v, sem_g).wait()
                pltpu.async_copy(rows_v, out_hbm.at[pl.ds(base, CHUNK)], sem_s)
            return carry

        lax.fori_loop(0, n_chunks // 2, body, 0)
        for rows_v, _, sem_s in bufs:
            pltpu.make_async_copy(
                rows_v, out_hbm.at[pl.ds(base0, CHUNK)], sem_s).wait()

    return k(fused, idx)


def kernel(inputs, token_table, pos_table):
    batch = inputs.shape[0]
    flat = inputs.reshape(batch, SEQ)
    fused3, idx = _prep_tc(flat, token_table, pos_table)
    out2 = _gather_sc(fused3.reshape(FUSED_ROWS, EMBED), idx.reshape(-1))
    return out2.reshape(batch, SEQ, EMBED)


# speedup vs baseline: 15.2344x; 1.0365x over previous
import functools

import jax
import jax.numpy as jnp
from jax import lax
from jax.experimental import pallas as pl
from jax.experimental.pallas import tpu as pltpu
from jax.experimental.pallas import tpu_sc as plsc

EMBED = 128
VOCAB = 14
SEQ = 64
FUSED_ROWS = SEQ * VOCAB
NC = 2
NS = 16
NW = NC * NS
GCHUNK = 128
GPER = 2
CHUNK = GCHUNK * GPER


def _prep_tc(flat, token_table, pos_table):
    b = flat.shape[0]

    def body(flat_ref, tok_ref, pos_ref, f_ref, idx_ref):
        f_ref[...] = pos_ref[...][:, None, :] + tok_ref[...][None, :, :]
        pos_ids = lax.broadcasted_iota(jnp.int32, (b, SEQ), 1)
        idx_ref[...] = flat_ref[...] + pos_ids * VOCAB

    return pl.pallas_call(
        body,
        out_shape=(
            jax.ShapeDtypeStruct((SEQ, VOCAB, EMBED), jnp.float32),
            jax.ShapeDtypeStruct((b, SEQ), jnp.int32),
        ),
    )(flat, token_table, pos_table)


def _gather_sc(fused, idx):
    n_rows = idx.shape[0]
    rows_per_w = n_rows // NW
    n_chunks = rows_per_w // CHUNK
    mesh = plsc.VectorSubcoreMesh(core_axis_name="c", subcore_axis_name="s")

    @functools.partial(
        pl.kernel,
        out_type=jax.ShapeDtypeStruct((n_rows, EMBED), jnp.float32),
        mesh=mesh,
        scratch_types=[
            pltpu.VMEM((rows_per_w,), jnp.int32),
            pltpu.VMEM((CHUNK, EMBED), jnp.float32),
            pltpu.VMEM((CHUNK, EMBED), jnp.float32),
            pltpu.VMEM_SHARED((FUSED_ROWS, EMBED), jnp.float32),
            pltpu.SemaphoreType.DMA,
            pltpu.SemaphoreType.DMA,
            pltpu.SemaphoreType.DMA,
            pltpu.SemaphoreType.DMA,
        ],
    )
    def k(fused_hbm, idx_hbm, out_hbm, idx_all, rows_v0, rows_v1, fused_sp,
          sem_g0, sem_g1, sem_s0, sem_s1):
        sid = lax.axis_index("s")
        wid = sid * NC + lax.axis_index("c")
        base0 = wid * rows_per_w

        @pl.when(sid == 0)
        def _():
            pltpu.sync_copy(fused_hbm, fused_sp)

        pltpu.sync_copy(idx_hbm.at[pl.ds(base0, rows_per_w)], idx_all)
        plsc.subcore_barrier()

        bufs = ((rows_v0, sem_g0, sem_s0), (rows_v1, sem_g1, sem_s1))

        def body(j, carry):
            for b, (rows_v, sem_g, sem_s) in enumerate(bufs):
                i = j * 2 + b
                base = base0 + i * CHUNK

                @pl.when(j > 0)
                def _():
                    pltpu.make_async_copy(
                        rows_v, out_hbm.at[pl.ds(base0, CHUNK)], sem_s).wait()

                handles = [
                    pltpu.async_copy(
                        fused_sp.at[idx_all.at[
                            pl.ds(i * CHUNK + g * GCHUNK, GCHUNK)]],
                        rows_v.at[pl.ds(g * GCHUNK, GCHUNK)], sem_g)
                    for g in range(GPER)
                ]
                for h in handles:
                    h.wait()
                pltpu.async_copy(rows_v, out_hbm.at[pl.ds(base, CHUNK)], sem_s)
            return carry

        lax.fori_loop(0, n_chunks // 2, body, 0)
        for rows_v, _, sem_s in bufs:
            pltpu.make_async_copy(
                rows_v, out_hbm.at[pl.ds(base0, CHUNK)], sem_s).wait()

    return k(fused, idx)


def kernel(inputs, token_table, pos_table):
    batch = inputs.shape[0]
    flat = inputs.reshape(batch, SEQ)
    fused3, idx = _prep_tc(flat, token_table, pos_table)
    out2 = _gather_sc(fused3.reshape(FUSED_ROWS, EMBED), idx.reshape(-1))
    return out2.reshape(batch, SEQ, EMBED)
